# baseline (speedup 1.0000x reference)
import functools

import jax
import jax.numpy as jnp
from jax import lax
from jax.experimental import pallas as pl
from jax.experimental.pallas import tpu as pltpu
from jax.experimental.pallas import tpu_sc as plsc

DIM = 64
NC = 2
NS = 16
NW = NC * NS
CHUNK = 128


@functools.lru_cache(maxsize=None)
def _build(cpw: int):
    mesh = plsc.VectorSubcoreMesh(core_axis_name="c", subcore_axis_name="s")

    @functools.partial(
        pl.kernel,
        mesh=mesh,
        out_type=jax.ShapeDtypeStruct((NW * cpw, CHUNK, DIM), jnp.float32),
        compiler_params=pltpu.CompilerParams(use_tc_tiling_on_sc=False),
        scratch_types=[
            pltpu.VMEM((cpw, CHUNK), jnp.int32),
            pltpu.VMEM((CHUNK, DIM), jnp.float32),
            pltpu.SemaphoreType.DMA,
        ],
    )
    def gather_kernel(table_hbm, idx_hbm, out_hbm, idx_v, rows_v, gsem):
        wid = lax.axis_index("s") * NC + lax.axis_index("c")
        pltpu.sync_copy(idx_hbm.at[wid], idx_v)

        def step(j, carry):
            pltpu.async_copy(table_hbm.at[idx_v.at[j]], rows_v, gsem).wait()
            pltpu.sync_copy(rows_v, out_hbm.at[wid * cpw + j])
            return carry

        lax.fori_loop(0, cpw, step, 0)

    return gather_kernel


def kernel(text, table):
    B = text.size
    assert B % (NW * CHUNK) == 0
    cpw = B // (NW * CHUNK)
    idx = text.reshape(NW, cpw, CHUNK).astype(jnp.int32)
    out = _build(cpw)(table, idx)
    return out.reshape(text.shape + (DIM,))


# speedup vs baseline: 1.0459x; 1.0459x over previous
import functools

import jax
import jax.numpy as jnp
from jax import lax
from jax.experimental import pallas as pl
from jax.experimental.pallas import tpu as pltpu
from jax.experimental.pallas import tpu_sc as plsc

DIM = 64
NC = 2
NS = 16
NW = NC * NS
CHUNK = 128
NBUF = 5


@functools.lru_cache(maxsize=None)
def _build(cpw: int):
    assert cpw % NBUF == 0 and cpw // NBUF >= 2
    n_main_groups = cpw // NBUF - 1
    mesh = plsc.VectorSubcoreMesh(core_axis_name="c", subcore_axis_name="s")

    @functools.partial(
        pl.kernel,
        mesh=mesh,
        out_type=jax.ShapeDtypeStruct((NW * cpw, CHUNK, DIM), jnp.float32),
        compiler_params=pltpu.CompilerParams(use_tc_tiling_on_sc=False),
        scratch_types=[
            pltpu.VMEM((cpw, CHUNK), jnp.int32),
            pltpu.VMEM((NBUF, CHUNK, DIM), jnp.float32),
            pltpu.SemaphoreType.DMA((NBUF,)),
            pltpu.SemaphoreType.DMA((NBUF,)),
        ],
    )
    def gather_kernel(table_hbm, idx_hbm, out_hbm, idx_v, rows_v, gsem, wsem):
        wid = lax.axis_index("s") * NC + lax.axis_index("c")
        pltpu.sync_copy(idx_hbm.at[wid], idx_v)
        base = wid * cpw

        for b in range(NBUF):
            pltpu.async_copy(table_hbm.at[idx_v.at[b]], rows_v.at[b], gsem.at[b])

        def group(g, carry):
            j0 = g * NBUF
            for b in range(NBUF):
                j = j0 + b
                pltpu.make_async_copy(
                    table_hbm.at[idx_v.at[j]], rows_v.at[b], gsem.at[b]
                ).wait()
                w = pltpu.async_copy(rows_v.at[b], out_hbm.at[base + j], wsem.at[b])
                w.wait()
                pltpu.async_copy(
                    table_hbm.at[idx_v.at[j + NBUF]], rows_v.at[b], gsem.at[b]
                )
            return carry

        lax.fori_loop(0, n_main_groups, group, 0)

        j0 = n_main_groups * NBUF
        for b in range(NBUF):
            j = j0 + b
            pltpu.make_async_copy(
                table_hbm.at[idx_v.at[j]], rows_v.at[b], gsem.at[b]
            ).wait()
            pltpu.async_copy(rows_v.at[b], out_hbm.at[base + j], wsem.at[b])
        for b in range(NBUF):
            j = j0 + b
            pltpu.make_async_copy(
                rows_v.at[b], out_hbm.at[base + j], wsem.at[b]
            ).wait()

    return gather_kernel


def kernel(text, table):
    B = text.size
    assert B % (NW * CHUNK) == 0
    cpw = B // (NW * CHUNK)
    idx = text.reshape(NW, cpw, CHUNK).astype(jnp.int32)
    out = _build(cpw)(table, idx)
    return out.reshape(text.shape + (DIM,))


# speedup vs baseline: 1.0477x; 1.0017x over previous
import functools

import jax
import jax.numpy as jnp
from jax import lax
from jax.experimental import pallas as pl
from jax.experimental.pallas import tpu as pltpu
from jax.experimental.pallas import tpu_sc as plsc

DIM = 64
NC = 2
NS = 16
NW = NC * NS
SPC = 1
NBUF = 8


@functools.lru_cache(maxsize=None)
def _build(n_sent: int, seq: int):
    spw = n_sent // NW
    cpw = spw // SPC
    assert cpw % NBUF == 0 and cpw // NBUF >= 2
    n_main_groups = cpw // NBUF - 1
    mesh = plsc.VectorSubcoreMesh(core_axis_name="c", subcore_axis_name="s")

    @functools.partial(
        pl.kernel,
        mesh=mesh,
        out_type=jax.ShapeDtypeStruct((n_sent, seq, DIM), jnp.float32),
        compiler_params=pltpu.CompilerParams(use_tc_tiling_on_sc=False),
        scratch_types=[
            pltpu.VMEM((spw, seq), jnp.int32),
            pltpu.VMEM((NBUF, seq, DIM), jnp.float32),
            pltpu.SemaphoreType.DMA((NBUF,)),
            pltpu.SemaphoreType.DMA((NBUF,)),
        ],
    )
    def gather_kernel(table_hbm, idx_hbm, out_hbm, idx_v, rows_v, gsem, wsem):
        wid = lax.axis_index("s") * NC + lax.axis_index("c")
        base = wid * spw
        pltpu.sync_copy(idx_hbm.at[pl.ds(base, spw)], idx_v)

        for b in range(NBUF):
            pltpu.async_copy(table_hbm.at[idx_v.at[b]], rows_v.at[b], gsem.at[b])

        def group(g, carry):
            j0 = g * NBUF
            for b in range(NBUF):
                j = j0 + b
                pltpu.make_async_copy(
                    table_hbm.at[idx_v.at[j]], rows_v.at[b], gsem.at[b]
                ).wait()
                w = pltpu.async_copy(
                    rows_v.at[b], out_hbm.at[base + j], wsem.at[b]
                )
                w.wait()
                pltpu.async_copy(
                    table_hbm.at[idx_v.at[j + NBUF]], rows_v.at[b], gsem.at[b]
                )
            return carry

        lax.fori_loop(0, n_main_groups, group, 0)

        j0 = n_main_groups * NBUF
        for b in range(NBUF):
            j = j0 + b
            pltpu.make_async_copy(
                table_hbm.at[idx_v.at[j]], rows_v.at[b], gsem.at[b]
            ).wait()
            pltpu.async_copy(rows_v.at[b], out_hbm.at[base + j], wsem.at[b])
        for b in range(NBUF):
            j = j0 + b
            pltpu.make_async_copy(
                rows_v.at[b], out_hbm.at[base + j], wsem.at[b]
            ).wait()

    return gather_kernel


def kernel(text, table):
    n_sent, seq = text.shape
    out = _build(n_sent, seq)(table, text.astype(jnp.int32))
    return out
